# Initial kernel scaffold; baseline (speedup 1.0000x reference)
import functools

import jax
import jax.numpy as jnp
from jax import lax
from jax.experimental import pallas as pl
from jax.experimental.pallas import tpu as pltpu
from jax.experimental.pallas import tpu_sc as plsc

_T, _N, _E, _B = 4, 10000, 320000, 16
_H, _LH, _NCLS = 256, 128, 3

_NW = 32
_CPT = 4
_ROWS_PER_PASS = _NW * _CPT
_NPASS = _H // _ROWS_PER_PASS
_CH = 4000
_NCHUNK = _E // _CH
_EPT = _E // _NW

_mesh = plsc.VectorSubcoreMesh(core_axis_name="c", subcore_axis_name="s")


def _wid():
    return lax.axis_index("c") * 16 + lax.axis_index("s")


@functools.partial(
    pl.kernel,
    out_type=jax.ShapeDtypeStruct((_T * _NW, _N), jnp.float32),
    mesh=_mesh,
    scratch_types=[
        pltpu.VMEM((_EPT,), jnp.int32),
        pltpu.VMEM((_N,), jnp.float32),
    ],
)
def _deg_kernel(pk_hbm, out_hbm, pk_v, deg_v):
    w = _wid()
    ones = jnp.ones((16,), jnp.float32)
    zeros = jnp.zeros((16,), jnp.float32)
    for t in range(_T):
        def zbody(i, c):
            deg_v[pl.ds(i * 16, 16)] = zeros
            return c
        lax.fori_loop(0, _N // 16, zbody, 0)
        pltpu.sync_copy(pk_hbm.at[t * _NW + w], pk_v)

        def body(i, c):
            p16 = pk_v[pl.ds(i * 16, 16)]
            d = lax.shift_right_logical(p16, 16)
            plsc.addupdate_scatter(deg_v, [d], ones)
            return c
        lax.fori_loop(0, _EPT // 16, body, 0)
        pltpu.sync_copy(deg_v, out_hbm.at[t * _NW + w])


@functools.partial(
    pl.kernel,
    out_type=jax.ShapeDtypeStruct((_H, _N), jnp.float32),
    mesh=_mesh,
    scratch_types=[
        pltpu.VMEM((_CPT, _N), jnp.float32),
        pltpu.VMEM((_CPT, _N), jnp.float32),
        pltpu.VMEM((_CH,), jnp.int32),
        pltpu.VMEM((_CH,), jnp.int32),
        pltpu.SemaphoreType.DMA,
        pltpu.SemaphoreType.DMA,
    ],
)
def _agg_kernel(y_hbm, pk_hbm, out_hbm, y_v, agg_v, pk0_v, pk1_v, sem0, sem1):
    w = _wid()
    cvecs = [jnp.full((16,), c, jnp.int32) for c in range(_CPT)]
    mask = jnp.int32(0xFFFF)
    bufs = ((pk0_v, sem0), (pk1_v, sem1))
    for p in range(_NPASS):
        row0 = p * _ROWS_PER_PASS + w * _CPT
        pltpu.sync_copy(y_hbm.at[pl.ds(row0, _CPT)], y_v)
        pltpu.sync_copy(y_hbm.at[pl.ds(row0, _CPT)], agg_v)
        pltpu.make_async_copy(pk_hbm.at[0], pk0_v, sem0).start()
        pltpu.make_async_copy(pk_hbm.at[1], pk1_v, sem1).start()

        def outer(g, c):
            for b in range(2):
                pkb, semb = bufs[b]
                gi = g * 2 + b
                pltpu.make_async_copy(pk_hbm.at[gi], pkb, semb).wait()

                def inner(i, cc):
                    p16 = pkb[pl.ds(i * 16, 16)]
                    s = jnp.bitwise_and(p16, mask)
                    d = lax.shift_right_logical(p16, 16)
                    for c4 in range(_CPT):
                        v = plsc.load_gather(y_v, [cvecs[c4], s])
                        plsc.addupdate_scatter(agg_v, [cvecs[c4], d], v)
                    return cc
                lax.fori_loop(0, _CH // 16, inner, 0, unroll=2)

                @pl.when(gi + 2 < _NCHUNK)
                def _start_next():
                    pltpu.make_async_copy(pk_hbm.at[gi + 2], pkb, semb).start()
            return c
        lax.fori_loop(0, _NCHUNK // 2, outer, 0)
        pltpu.sync_copy(agg_v, out_hbm.at[pl.ds(row0, _CPT)])


def _mm(a, b):
    return lax.dot_general(a, b, (((1,), (0,)), ((), ())),
                           preferred_element_type=jnp.float32)


def _bn_relu(agg, dis, b, g, be):
    z = agg * dis + b
    m = jnp.mean(z, axis=1, keepdims=True)
    zc = z - m
    v = jnp.mean(zc * zc, axis=1, keepdims=True)
    xh = zc * lax.rsqrt(v + 1e-5) * g + be
    return jnp.maximum(xh, 0.0)


def _k1_body(poss_ref, ids_ref, e0_ref, e1_ref, e2_ref, e5_ref, w1t_ref,
             degp_ref, y_ref, dis_ref):
    deg = jnp.sum(degp_ref[...], axis=0, keepdims=True) + 1.0
    dis = lax.rsqrt(deg)
    dis_ref[...] = dis
    ids = ids_ref[...]
    parts = [poss_ref[...]]
    for row, tref in ((0, e0_ref), (1, e1_ref), (2, e2_ref), (3, e5_ref)):
        tbl = tref[...]
        k = tbl.shape[1]
        iota = lax.broadcasted_iota(jnp.int32, (k, _N), 0)
        oh = (iota == ids[row:row + 1, :]).astype(jnp.float32)
        parts.append(_mm(tbl, oh))
    x0 = jnp.concatenate(parts, axis=0)
    y_ref[...] = _mm(w1t_ref[...], x0) * dis


def _layer_body(agg_ref, dis_ref, b_ref, g_ref, be_ref, wt_ref, y_ref):
    dis = dis_ref[...]
    x = _bn_relu(agg_ref[...], dis, b_ref[...], g_ref[...], be_ref[...])
    y_ref[...] = _mm(wt_ref[...], x) * dis


def _pool_body(agg_ref, dis_ref, b_ref, g_ref, be_ref, batch_ref, feat_ref):
    x = _bn_relu(agg_ref[...], dis_ref[...], b_ref[...], g_ref[...],
                 be_ref[...])
    bt = batch_ref[...]
    iota = lax.broadcasted_iota(jnp.int32, (_N, _B), 1)
    ob = (iota == bt).astype(jnp.float32)
    sums = _mm(x, ob)
    cnt = jnp.sum(ob, axis=0, keepdims=True)
    feat_ref[...] = sums / jnp.maximum(cnt, 1.0)


def _head_body(seq_ref, wih0_ref, whh0_ref, bb0_ref, wih1_ref, whh1_ref,
               bb1_ref, fc1t_ref, fc1b_ref, fc2t_ref, fc2b_ref, out_ref):
    def lstm(xs, wih, whh, bb):
        h = jnp.zeros((_LH, _B), jnp.float32)
        c = jnp.zeros((_LH, _B), jnp.float32)
        outs = []
        for t in range(_T):
            gt = _mm(wih, xs[t]) + _mm(whh, h) + bb
            i = jax.nn.sigmoid(gt[0:_LH])
            f = jax.nn.sigmoid(gt[_LH:2 * _LH])
            gg = jnp.tanh(gt[2 * _LH:3 * _LH])
            o = jax.nn.sigmoid(gt[3 * _LH:4 * _LH])
            c = f * c + i * gg
            h = o * jnp.tanh(c)
            outs.append(h)
        return outs

    seq = seq_ref[...]
    o1 = lstm([seq[t] for t in range(_T)], wih0_ref[...], whh0_ref[...],
              bb0_ref[...])
    o2 = lstm(o1, wih1_ref[...], whh1_ref[...], bb1_ref[...])
    last = o2[-1]
    hcl = jnp.maximum(_mm(fc1t_ref[...], last) + fc1b_ref[...], 0.0)
    out_ref[...] = _mm(fc2t_ref[...], hcl) + fc2b_ref[...]


def _tc(body, out_shape, *args):
    return pl.pallas_call(body, out_shape=out_shape)(*args)


def kernel(possibility, node_ids, edge_index, batch, emb0, emb1, emb2, emb5,
           W1, b1, W2, b2, W3, b3, g1, be1, g2, be2, g3, be3,
           Wih0, Whh0, bih0, bhh0, Wih1, Whh1, bih1, bhh1,
           fc1_W, fc1_b, fc2_W, fc2_b):
    f32 = jnp.float32
    sds = jax.ShapeDtypeStruct
    possT = jnp.transpose(possibility, (0, 2, 1))
    idsT = jnp.transpose(node_ids, (0, 2, 1)).astype(jnp.int32)
    ei = edge_index.astype(jnp.int32)
    pk = jnp.bitwise_or(ei[:, 0, :], jnp.left_shift(ei[:, 1, :], 16))
    pk_deg = pk.reshape(_T * _NW, _EPT)
    pk_chunks = pk.reshape(_T, _NCHUNK, _CH)
    batch_col = batch.astype(jnp.int32).reshape(_T, _N, 1)
    e0t, e1t, e2t, e5t = (jnp.transpose(e) for e in (emb0, emb1, emb2, emb5))
    w1t, w2t, w3t = (jnp.transpose(w) for w in (W1, W2, W3))
    col = lambda v: v.reshape(-1, 1)

    degp = _deg_kernel(pk_deg)

    feats = []
    for t in range(_T):
        y1, dis = _tc(_k1_body,
                      (sds((_H, _N), f32), sds((1, _N), f32)),
                      possT[t], idsT[t], e0t, e1t, e2t, e5t, w1t,
                      degp[t * _NW:(t + 1) * _NW])
        a1 = _agg_kernel(y1, pk_chunks[t])
        y2 = _tc(_layer_body, sds((_H, _N), f32),
                 a1, dis, col(b1), col(g1), col(be1), w2t)
        a2 = _agg_kernel(y2, pk_chunks[t])
        y3 = _tc(_layer_body, sds((_H, _N), f32),
                 a2, dis, col(b2), col(g2), col(be2), w3t)
        a3 = _agg_kernel(y3, pk_chunks[t])
        ft = _tc(_pool_body, sds((_H, _B), f32),
                 a3, dis, col(b3), col(g3), col(be3), batch_col[t])
        feats.append(ft)

    seq = jnp.stack(feats, 0)
    outT = _tc(_head_body, sds((_NCLS, _B), f32),
               seq, Wih0, Whh0, col(bih0) + col(bhh0),
               Wih1, Whh1, col(bih1) + col(bhh1),
               jnp.transpose(fc1_W), col(fc1_b),
               jnp.transpose(fc2_W), col(fc2_b))
    return jnp.transpose(outT)


# speedup vs baseline: 3.5557x; 3.5557x over previous
import functools

import jax
import jax.numpy as jnp
from jax import lax
from jax.experimental import pallas as pl
from jax.experimental.pallas import tpu as pltpu
from jax.experimental.pallas import tpu_sc as plsc

_T, _N, _E, _B = 4, 10000, 320000, 16
_H, _LH, _NCLS = 256, 128, 3

_NW = 32
_CPT = 4
_ROWS_PER_PASS = _NW * _CPT
_NPASS = _H // _ROWS_PER_PASS
_CH = 4000
_NCHUNK = _E // _CH
_EPT = _E // _NW

_mesh = plsc.VectorSubcoreMesh(core_axis_name="c", subcore_axis_name="s")


def _wid():
    return lax.axis_index("c") * 16 + lax.axis_index("s")


@functools.partial(
    pl.kernel,
    out_type=jax.ShapeDtypeStruct((_T * _NW, _N), jnp.float32),
    mesh=_mesh,
    scratch_types=[
        pltpu.VMEM((_EPT,), jnp.int32),
        pltpu.VMEM((_N,), jnp.float32),
    ],
    compiler_params=pltpu.CompilerParams(needs_layout_passes=False),
)
def _deg_kernel(pk_hbm, out_hbm, pk_v, deg_v):
    w = _wid()
    ones = jnp.ones((16,), jnp.float32)
    zeros = jnp.zeros((16,), jnp.float32)
    for t in range(_T):
        def zbody(i, c):
            deg_v[pl.ds(i * 16, 16)] = zeros
            return c
        lax.fori_loop(0, _N // 16, zbody, 0)
        pltpu.sync_copy(pk_hbm.at[t * _NW + w], pk_v)

        def body(i, c):
            p16 = pk_v[pl.ds(i * 16, 16)]
            d = lax.shift_right_logical(p16, 16)
            plsc.addupdate_scatter(deg_v, [d], ones)
            return c
        lax.fori_loop(0, _EPT // 16, body, 0)
        pltpu.sync_copy(deg_v, out_hbm.at[t * _NW + w])


@functools.partial(
    pl.kernel,
    out_type=jax.ShapeDtypeStruct((_H, _N), jnp.float32),
    mesh=_mesh,
    scratch_types=[
        pltpu.VMEM((_CPT, _N), jnp.float32),
        pltpu.VMEM((_CPT, _N), jnp.float32),
        pltpu.VMEM((_CH,), jnp.int32),
        pltpu.VMEM((_CH,), jnp.int32),
        pltpu.SemaphoreType.DMA,
        pltpu.SemaphoreType.DMA,
    ],
    compiler_params=pltpu.CompilerParams(needs_layout_passes=False),
)
def _agg_kernel(y_hbm, pk_hbm, out_hbm, y_v, agg_v, pk0_v, pk1_v, sem0, sem1):
    w = _wid()
    cvecs = [jnp.full((16,), c, jnp.int32) for c in range(_CPT)]
    mask = jnp.int32(0xFFFF)
    bufs = ((pk0_v, sem0), (pk1_v, sem1))
    for p in range(_NPASS):
        row0 = p * _ROWS_PER_PASS + w * _CPT
        pltpu.sync_copy(y_hbm.at[pl.ds(row0, _CPT)], y_v)
        pltpu.sync_copy(y_hbm.at[pl.ds(row0, _CPT)], agg_v)
        pltpu.make_async_copy(pk_hbm.at[0], pk0_v, sem0).start()
        pltpu.make_async_copy(pk_hbm.at[1], pk1_v, sem1).start()

        def outer(g, c):
            for b in range(2):
                pkb, semb = bufs[b]
                gi = g * 2 + b
                pltpu.make_async_copy(pk_hbm.at[gi], pkb, semb).wait()

                def inner(i, cc):
                    p16 = pkb[pl.ds(i * 16, 16)]
                    s = jnp.bitwise_and(p16, mask)
                    d = lax.shift_right_logical(p16, 16)
                    for c4 in range(_CPT):
                        v = plsc.load_gather(y_v, [cvecs[c4], s])
                        plsc.addupdate_scatter(agg_v, [cvecs[c4], d], v)
                    return cc
                lax.fori_loop(0, _CH // 16, inner, 0, unroll=2)

                @pl.when(gi + 2 < _NCHUNK)
                def _start_next():
                    pltpu.make_async_copy(pk_hbm.at[gi + 2], pkb, semb).start()
            return c
        lax.fori_loop(0, _NCHUNK // 2, outer, 0)
        pltpu.sync_copy(agg_v, out_hbm.at[pl.ds(row0, _CPT)])


def _mm(a, b):
    return lax.dot_general(a, b, (((1,), (0,)), ((), ())),
                           preferred_element_type=jnp.float32)


def _bn_relu(agg, dis, b, g, be):
    z = agg * dis + b
    m = jnp.mean(z, axis=1, keepdims=True)
    zc = z - m
    v = jnp.mean(zc * zc, axis=1, keepdims=True)
    xh = zc * lax.rsqrt(v + 1e-5) * g + be
    return jnp.maximum(xh, 0.0)


def _k1_body(poss_ref, ids_ref, e0_ref, e1_ref, e2_ref, e5_ref, w1t_ref,
             degp_ref, y_ref, dis_ref):
    deg = jnp.sum(degp_ref[...], axis=0, keepdims=True) + 1.0
    dis = lax.rsqrt(deg)
    dis_ref[...] = dis
    ids = ids_ref[...]
    parts = [poss_ref[...]]
    for row, tref in ((0, e0_ref), (1, e1_ref), (2, e2_ref), (3, e5_ref)):
        tbl = tref[...]
        k = tbl.shape[1]
        iota = lax.broadcasted_iota(jnp.int32, (k, _N), 0)
        oh = (iota == ids[row:row + 1, :]).astype(jnp.float32)
        parts.append(_mm(tbl, oh))
    x0 = jnp.concatenate(parts, axis=0)
    y_ref[...] = _mm(w1t_ref[...], x0) * dis


def _layer_body(agg_ref, dis_ref, b_ref, g_ref, be_ref, wt_ref, y_ref):
    dis = dis_ref[...]
    x = _bn_relu(agg_ref[...], dis, b_ref[...], g_ref[...], be_ref[...])
    y_ref[...] = _mm(wt_ref[...], x) * dis


def _pool_body(agg_ref, dis_ref, b_ref, g_ref, be_ref, batch_ref, feat_ref):
    x = _bn_relu(agg_ref[...], dis_ref[...], b_ref[...], g_ref[...],
                 be_ref[...])
    bt = batch_ref[...]
    iota = lax.broadcasted_iota(jnp.int32, (_N, _B), 1)
    ob = (iota == bt).astype(jnp.float32)
    sums = _mm(x, ob)
    cnt = jnp.sum(ob, axis=0, keepdims=True)
    feat_ref[...] = sums / jnp.maximum(cnt, 1.0)


def _head_body(seq_ref, wih0_ref, whh0_ref, bb0_ref, wih1_ref, whh1_ref,
               bb1_ref, fc1t_ref, fc1b_ref, fc2t_ref, fc2b_ref, out_ref):
    def lstm(xs, wih, whh, bb):
        h = jnp.zeros((_LH, _B), jnp.float32)
        c = jnp.zeros((_LH, _B), jnp.float32)
        outs = []
        for t in range(_T):
            gt = _mm(wih, xs[t]) + _mm(whh, h) + bb
            i = jax.nn.sigmoid(gt[0:_LH])
            f = jax.nn.sigmoid(gt[_LH:2 * _LH])
            gg = jnp.tanh(gt[2 * _LH:3 * _LH])
            o = jax.nn.sigmoid(gt[3 * _LH:4 * _LH])
            c = f * c + i * gg
            h = o * jnp.tanh(c)
            outs.append(h)
        return outs

    seq = seq_ref[...]
    o1 = lstm([seq[t] for t in range(_T)], wih0_ref[...], whh0_ref[...],
              bb0_ref[...])
    o2 = lstm(o1, wih1_ref[...], whh1_ref[...], bb1_ref[...])
    last = o2[-1]
    hcl = jnp.maximum(_mm(fc1t_ref[...], last) + fc1b_ref[...], 0.0)
    out_ref[...] = _mm(fc2t_ref[...], hcl) + fc2b_ref[...]


def _tc(body, out_shape, *args):
    return pl.pallas_call(body, out_shape=out_shape)(*args)


def kernel(possibility, node_ids, edge_index, batch, emb0, emb1, emb2, emb5,
           W1, b1, W2, b2, W3, b3, g1, be1, g2, be2, g3, be3,
           Wih0, Whh0, bih0, bhh0, Wih1, Whh1, bih1, bhh1,
           fc1_W, fc1_b, fc2_W, fc2_b):
    f32 = jnp.float32
    sds = jax.ShapeDtypeStruct
    possT = jnp.transpose(possibility, (0, 2, 1))
    idsT = jnp.transpose(node_ids, (0, 2, 1)).astype(jnp.int32)
    ei = edge_index.astype(jnp.int32)
    pk = jnp.bitwise_or(ei[:, 0, :], jnp.left_shift(ei[:, 1, :], 16))
    pk_deg = pk.reshape(_T * _NW, _EPT)
    pk_chunks = pk.reshape(_T, _NCHUNK, _CH)
    batch_col = batch.astype(jnp.int32).reshape(_T, _N, 1)
    e0t, e1t, e2t, e5t = (jnp.transpose(e) for e in (emb0, emb1, emb2, emb5))
    w1t, w2t, w3t = (jnp.transpose(w) for w in (W1, W2, W3))
    col = lambda v: v.reshape(-1, 1)

    degp = _deg_kernel(pk_deg)

    feats = []
    for t in range(_T):
        y1, dis = _tc(_k1_body,
                      (sds((_H, _N), f32), sds((1, _N), f32)),
                      possT[t], idsT[t], e0t, e1t, e2t, e5t, w1t,
                      degp[t * _NW:(t + 1) * _NW])
        a1 = _agg_kernel(y1, pk_chunks[t])
        y2 = _tc(_layer_body, sds((_H, _N), f32),
                 a1, dis, col(b1), col(g1), col(be1), w2t)
        a2 = _agg_kernel(y2, pk_chunks[t])
        y3 = _tc(_layer_body, sds((_H, _N), f32),
                 a2, dis, col(b2), col(g2), col(be2), w3t)
        a3 = _agg_kernel(y3, pk_chunks[t])
        ft = _tc(_pool_body, sds((_H, _B), f32),
                 a3, dis, col(b3), col(g3), col(be3), batch_col[t])
        feats.append(ft)

    seq = jnp.stack(feats, 0)
    outT = _tc(_head_body, sds((_NCLS, _B), f32),
               seq, Wih0, Whh0, col(bih0) + col(bhh0),
               Wih1, Whh1, col(bih1) + col(bhh1),
               jnp.transpose(fc1_W), col(fc1_b),
               jnp.transpose(fc2_W), col(fc2_b))
    return jnp.transpose(outT)


# speedup vs baseline: 10.2479x; 2.8821x over previous
import functools

import jax
import jax.numpy as jnp
from jax import lax
from jax.experimental import pallas as pl
from jax.experimental.pallas import tpu as pltpu
from jax.experimental.pallas import tpu_sc as plsc

_T, _N, _E, _B = 4, 10000, 320000, 16
_H, _LH, _NCLS = 256, 128, 3

_NW = 32
_CPT = 4
_ROWS_PER_PASS = _NW * _CPT
_NPASS = _H // _ROWS_PER_PASS
_CH = 6400
_NCHUNK = _E // _CH
_EPT = _E // _NW

_mesh = plsc.VectorSubcoreMesh(core_axis_name="c", subcore_axis_name="s")


def _wid():
    return lax.axis_index("c") * 16 + lax.axis_index("s")


@functools.partial(
    pl.kernel,
    out_type=jax.ShapeDtypeStruct((_T * _NW, _N), jnp.float32),
    mesh=_mesh,
    scratch_types=[
        pltpu.VMEM((_EPT,), jnp.int32),
        pltpu.VMEM((_N,), jnp.float32),
    ],
    compiler_params=pltpu.CompilerParams(needs_layout_passes=False),
)
def _deg_kernel(pk_hbm, out_hbm, pk_v, deg_v):
    w = _wid()
    ones = jnp.ones((16,), jnp.float32)
    zeros = jnp.zeros((16,), jnp.float32)
    for t in range(_T):
        def zbody(i, c):
            deg_v[pl.ds(i * 16, 16)] = zeros
            return c
        lax.fori_loop(0, _N // 16, zbody, 0)
        pltpu.sync_copy(pk_hbm.at[t * _NW + w], pk_v)

        def body(i, c):
            p16 = pk_v[pl.ds(i * 16, 16)]
            d = lax.shift_right_logical(p16, 16)
            plsc.addupdate_scatter(deg_v, [d], ones)
            return c
        lax.fori_loop(0, _EPT // 16, body, 0)
        pltpu.sync_copy(deg_v, out_hbm.at[t * _NW + w])


@functools.partial(
    pl.kernel,
    out_type=jax.ShapeDtypeStruct((_H * _N,), jnp.float32),
    mesh=_mesh,
    scratch_types=[
        pltpu.VMEM((_CPT * _N,), jnp.float32),
        pltpu.VMEM((_CPT * _N,), jnp.float32),
        pltpu.VMEM((_CH,), jnp.int32),
        pltpu.VMEM((_CH,), jnp.int32),
        pltpu.SemaphoreType.DMA,
        pltpu.SemaphoreType.DMA,
    ],
    compiler_params=pltpu.CompilerParams(needs_layout_passes=False),
)
def _agg_kernel(y_hbm, pk_hbm, out_hbm, y_v, agg_v, pk0_v, pk1_v, sem0, sem1):
    w = _wid()
    mask = jnp.int32(0xFFFF)
    bufs = ((pk0_v, sem0), (pk1_v, sem1))
    for p in range(_NPASS):
        row0 = p * _ROWS_PER_PASS + w * _CPT
        pltpu.sync_copy(y_hbm.at[pl.ds(row0 * _N, _CPT * _N)], y_v)
        pltpu.sync_copy(y_hbm.at[pl.ds(row0 * _N, _CPT * _N)], agg_v)
        pltpu.make_async_copy(pk_hbm.at[0], pk0_v, sem0).start()
        pltpu.make_async_copy(pk_hbm.at[1], pk1_v, sem1).start()

        def outer(g, c):
            for b in range(2):
                pkb, semb = bufs[b]
                gi = g * 2 + b
                pltpu.make_async_copy(pk_hbm.at[gi], pkb, semb).wait()

                @plsc.parallel_loop(0, _CH // 16, unroll=8)
                def _inner(i):
                    p16 = pkb[pl.ds(i * 16, 16)]
                    s = jnp.bitwise_and(p16, mask)
                    d = lax.shift_right_logical(p16, 16)
                    for c4 in range(_CPT):
                        v = plsc.load_gather(y_v, [s + (c4 * _N)])
                        plsc.addupdate_scatter(agg_v, [d + (c4 * _N)], v)

                @pl.when(gi + 2 < _NCHUNK)
                def _start_next():
                    pltpu.make_async_copy(pk_hbm.at[gi + 2], pkb, semb).start()
            return c
        lax.fori_loop(0, _NCHUNK // 2, outer, 0)
        pltpu.sync_copy(agg_v, out_hbm.at[pl.ds(row0 * _N, _CPT * _N)])


def _mm(a, b):
    return lax.dot_general(a, b, (((1,), (0,)), ((), ())),
                           preferred_element_type=jnp.float32)


def _bn_relu(agg, dis, b, g, be):
    z = agg * dis + b
    m = jnp.mean(z, axis=1, keepdims=True)
    zc = z - m
    v = jnp.mean(zc * zc, axis=1, keepdims=True)
    xh = zc * lax.rsqrt(v + 1e-5) * g + be
    return jnp.maximum(xh, 0.0)


def _k1_body(poss_ref, ids_ref, e0_ref, e1_ref, e2_ref, e5_ref, w1t_ref,
             degp_ref, y_ref, dis_ref):
    deg = jnp.sum(degp_ref[...], axis=0, keepdims=True) + 1.0
    dis = lax.rsqrt(deg)
    dis_ref[...] = dis
    ids = ids_ref[...]
    parts = [poss_ref[...]]
    for row, tref in ((0, e0_ref), (1, e1_ref), (2, e2_ref), (3, e5_ref)):
        tbl = tref[...]
        k = tbl.shape[1]
        iota = lax.broadcasted_iota(jnp.int32, (k, _N), 0)
        oh = (iota == ids[row:row + 1, :]).astype(jnp.float32)
        parts.append(_mm(tbl, oh))
    x0 = jnp.concatenate(parts, axis=0)
    y_ref[...] = _mm(w1t_ref[...], x0) * dis


def _layer_body(agg_ref, dis_ref, b_ref, g_ref, be_ref, wt_ref, y_ref):
    dis = dis_ref[...]
    x = _bn_relu(agg_ref[...], dis, b_ref[...], g_ref[...], be_ref[...])
    y_ref[...] = _mm(wt_ref[...], x) * dis


def _pool_body(agg_ref, dis_ref, b_ref, g_ref, be_ref, batch_ref, feat_ref):
    x = _bn_relu(agg_ref[...], dis_ref[...], b_ref[...], g_ref[...],
                 be_ref[...])
    bt = batch_ref[...]
    iota = lax.broadcasted_iota(jnp.int32, (_N, _B), 1)
    ob = (iota == bt).astype(jnp.float32)
    sums = _mm(x, ob)
    cnt = jnp.sum(ob, axis=0, keepdims=True)
    feat_ref[...] = sums / jnp.maximum(cnt, 1.0)


def _head_body(seq_ref, wih0_ref, whh0_ref, bb0_ref, wih1_ref, whh1_ref,
               bb1_ref, fc1t_ref, fc1b_ref, fc2t_ref, fc2b_ref, out_ref):
    def lstm(xs, wih, whh, bb):
        h = jnp.zeros((_LH, _B), jnp.float32)
        c = jnp.zeros((_LH, _B), jnp.float32)
        outs = []
        for t in range(_T):
            gt = _mm(wih, xs[t]) + _mm(whh, h) + bb
            i = jax.nn.sigmoid(gt[0:_LH])
            f = jax.nn.sigmoid(gt[_LH:2 * _LH])
            gg = jnp.tanh(gt[2 * _LH:3 * _LH])
            o = jax.nn.sigmoid(gt[3 * _LH:4 * _LH])
            c = f * c + i * gg
            h = o * jnp.tanh(c)
            outs.append(h)
        return outs

    seq = seq_ref[...]
    o1 = lstm([seq[t] for t in range(_T)], wih0_ref[...], whh0_ref[...],
              bb0_ref[...])
    o2 = lstm(o1, wih1_ref[...], whh1_ref[...], bb1_ref[...])
    last = o2[-1]
    hcl = jnp.maximum(_mm(fc1t_ref[...], last) + fc1b_ref[...], 0.0)
    out_ref[...] = _mm(fc2t_ref[...], hcl) + fc2b_ref[...]


def _tc(body, out_shape, *args):
    return pl.pallas_call(body, out_shape=out_shape)(*args)


def kernel(possibility, node_ids, edge_index, batch, emb0, emb1, emb2, emb5,
           W1, b1, W2, b2, W3, b3, g1, be1, g2, be2, g3, be3,
           Wih0, Whh0, bih0, bhh0, Wih1, Whh1, bih1, bhh1,
           fc1_W, fc1_b, fc2_W, fc2_b):
    f32 = jnp.float32
    sds = jax.ShapeDtypeStruct
    possT = jnp.transpose(possibility, (0, 2, 1))
    idsT = jnp.transpose(node_ids, (0, 2, 1)).astype(jnp.int32)
    ei = edge_index.astype(jnp.int32)
    pk = jnp.bitwise_or(ei[:, 0, :], jnp.left_shift(ei[:, 1, :], 16))
    pk_deg = pk.reshape(_T * _NW, _EPT)
    pk_chunks = pk.reshape(_T, _NCHUNK, _CH)
    batch_col = batch.astype(jnp.int32).reshape(_T, _N, 1)
    e0t, e1t, e2t, e5t = (jnp.transpose(e) for e in (emb0, emb1, emb2, emb5))
    w1t, w2t, w3t = (jnp.transpose(w) for w in (W1, W2, W3))
    col = lambda v: v.reshape(-1, 1)

    degp = _deg_kernel(pk_deg)

    feats = []
    for t in range(_T):
        y1, dis = _tc(_k1_body,
                      (sds((_H, _N), f32), sds((1, _N), f32)),
                      possT[t], idsT[t], e0t, e1t, e2t, e5t, w1t,
                      degp[t * _NW:(t + 1) * _NW])
        a1 = _agg_kernel(y1.reshape(-1), pk_chunks[t]).reshape(_H, _N)
        y2 = _tc(_layer_body, sds((_H, _N), f32),
                 a1, dis, col(b1), col(g1), col(be1), w2t)
        a2 = _agg_kernel(y2.reshape(-1), pk_chunks[t]).reshape(_H, _N)
        y3 = _tc(_layer_body, sds((_H, _N), f32),
                 a2, dis, col(b2), col(g2), col(be2), w3t)
        a3 = _agg_kernel(y3.reshape(-1), pk_chunks[t]).reshape(_H, _N)
        ft = _tc(_pool_body, sds((_H, _B), f32),
                 a3, dis, col(b3), col(g3), col(be3), batch_col[t])
        feats.append(ft)

    seq = jnp.stack(feats, 0)
    outT = _tc(_head_body, sds((_NCLS, _B), f32),
               seq, Wih0, Whh0, col(bih0) + col(bhh0),
               Wih1, Whh1, col(bih1) + col(bhh1),
               jnp.transpose(fc1_W), col(fc1_b),
               jnp.transpose(fc2_W), col(fc2_b))
    return jnp.transpose(outT)


# speedup vs baseline: 10.4524x; 1.0200x over previous
import functools

import jax
import jax.numpy as jnp
from jax import lax
from jax.experimental import pallas as pl
from jax.experimental.pallas import tpu as pltpu
from jax.experimental.pallas import tpu_sc as plsc

_T, _N, _E, _B = 4, 10000, 320000, 16
_H, _LH, _NCLS = 256, 128, 3

_NW = 32
_CPT = 4
_ROWS_PER_PASS = _NW * _CPT
_NPASS = _H // _ROWS_PER_PASS
_CH = 6400
_NCHUNK = _E // _CH
_EPT = _E // _NW

_mesh = plsc.VectorSubcoreMesh(core_axis_name="c", subcore_axis_name="s")


def _wid():
    return lax.axis_index("c") * 16 + lax.axis_index("s")


@functools.partial(
    pl.kernel,
    out_type=jax.ShapeDtypeStruct((_T * _NW, _N), jnp.float32),
    mesh=_mesh,
    scratch_types=[
        pltpu.VMEM((_EPT,), jnp.int32),
        pltpu.VMEM((_N,), jnp.float32),
    ],
    compiler_params=pltpu.CompilerParams(needs_layout_passes=False),
)
def _deg_kernel(pk_hbm, out_hbm, pk_v, deg_v):
    w = _wid()
    ones = jnp.ones((16,), jnp.float32)
    zeros = jnp.zeros((16,), jnp.float32)
    for t in range(_T):
        def zbody(i, c):
            deg_v[pl.ds(i * 16, 16)] = zeros
            return c
        lax.fori_loop(0, _N // 16, zbody, 0)
        pltpu.sync_copy(pk_hbm.at[t * _NW + w], pk_v)

        def body(i, c):
            p16 = pk_v[pl.ds(i * 16, 16)]
            d = lax.shift_right_logical(p16, 16)
            plsc.addupdate_scatter(deg_v, [d], ones)
            return c
        lax.fori_loop(0, _EPT // 16, body, 0)
        pltpu.sync_copy(deg_v, out_hbm.at[t * _NW + w])


@functools.partial(
    pl.kernel,
    out_type=jax.ShapeDtypeStruct((_H * _N,), jnp.float32),
    mesh=_mesh,
    scratch_types=[
        pltpu.VMEM((_CPT * _N,), jnp.float32),
        pltpu.VMEM((_CPT * _N,), jnp.float32),
        pltpu.VMEM((_CH,), jnp.int32),
        pltpu.VMEM((_CH,), jnp.int32),
        pltpu.SemaphoreType.DMA,
        pltpu.SemaphoreType.DMA,
    ],
    compiler_params=pltpu.CompilerParams(needs_layout_passes=False),
)
def _agg_kernel(y_hbm, pk_hbm, out_hbm, y_v, agg_v, pk0_v, pk1_v, sem0, sem1):
    w = _wid()
    mask = jnp.int32(0xFFFF)
    bufs = ((pk0_v, sem0), (pk1_v, sem1))
    for p in range(_NPASS):
        row0 = p * _ROWS_PER_PASS + w * _CPT
        pltpu.sync_copy(y_hbm.at[pl.ds(row0 * _N, _CPT * _N)], y_v)
        pltpu.sync_copy(y_hbm.at[pl.ds(row0 * _N, _CPT * _N)], agg_v)
        pltpu.make_async_copy(pk_hbm.at[0], pk0_v, sem0).start()
        pltpu.make_async_copy(pk_hbm.at[1], pk1_v, sem1).start()

        def outer(g, c):
            for b in range(2):
                pkb, semb = bufs[b]
                gi = g * 2 + b
                pltpu.make_async_copy(pk_hbm.at[gi], pkb, semb).wait()

                @plsc.parallel_loop(0, _CH // 16, unroll=16)
                def _inner(i):
                    p16 = pkb[pl.ds(i * 16, 16)]
                    s = jnp.bitwise_and(p16, mask)
                    d = lax.shift_right_logical(p16, 16)
                    for c4 in range(_CPT):
                        v = plsc.load_gather(y_v, [s + (c4 * _N)])
                        plsc.addupdate_scatter(agg_v, [d + (c4 * _N)], v)

                @pl.when(gi + 2 < _NCHUNK)
                def _start_next():
                    pltpu.make_async_copy(pk_hbm.at[gi + 2], pkb, semb).start()
            return c
        lax.fori_loop(0, _NCHUNK // 2, outer, 0)
        pltpu.sync_copy(agg_v, out_hbm.at[pl.ds(row0 * _N, _CPT * _N)])


def _mm(a, b):
    return lax.dot_general(a, b, (((1,), (0,)), ((), ())),
                           preferred_element_type=jnp.float32)


def _bn_relu(agg, dis, b, g, be):
    z = agg * dis + b
    m = jnp.mean(z, axis=1, keepdims=True)
    zc = z - m
    v = jnp.mean(zc * zc, axis=1, keepdims=True)
    xh = zc * lax.rsqrt(v + 1e-5) * g + be
    return jnp.maximum(xh, 0.0)


def _k1_body(poss_ref, ids_ref, e0_ref, e1_ref, e2_ref, e5_ref, w1t_ref,
             degp_ref, y_ref, dis_ref):
    deg = jnp.sum(degp_ref[...], axis=0, keepdims=True) + 1.0
    dis = lax.rsqrt(deg)
    dis_ref[...] = dis
    ids = ids_ref[...]
    parts = [poss_ref[...]]
    for row, tref in ((0, e0_ref), (1, e1_ref), (2, e2_ref), (3, e5_ref)):
        tbl = tref[...]
        k = tbl.shape[1]
        iota = lax.broadcasted_iota(jnp.int32, (k, _N), 0)
        oh = (iota == ids[row:row + 1, :]).astype(jnp.float32)
        parts.append(_mm(tbl, oh))
    x0 = jnp.concatenate(parts, axis=0)
    y_ref[...] = _mm(w1t_ref[...], x0) * dis


def _layer_body(agg_ref, dis_ref, b_ref, g_ref, be_ref, wt_ref, y_ref):
    dis = dis_ref[...]
    x = _bn_relu(agg_ref[...], dis, b_ref[...], g_ref[...], be_ref[...])
    y_ref[...] = _mm(wt_ref[...], x) * dis


def _pool_body(agg_ref, dis_ref, b_ref, g_ref, be_ref, batch_ref, feat_ref):
    x = _bn_relu(agg_ref[...], dis_ref[...], b_ref[...], g_ref[...],
                 be_ref[...])
    bt = batch_ref[...]
    iota = lax.broadcasted_iota(jnp.int32, (_N, _B), 1)
    ob = (iota == bt).astype(jnp.float32)
    sums = _mm(x, ob)
    cnt = jnp.sum(ob, axis=0, keepdims=True)
    feat_ref[...] = sums / jnp.maximum(cnt, 1.0)


def _head_body(seq_ref, wih0_ref, whh0_ref, bb0_ref, wih1_ref, whh1_ref,
               bb1_ref, fc1t_ref, fc1b_ref, fc2t_ref, fc2b_ref, out_ref):
    def lstm(xs, wih, whh, bb):
        h = jnp.zeros((_LH, _B), jnp.float32)
        c = jnp.zeros((_LH, _B), jnp.float32)
        outs = []
        for t in range(_T):
            gt = _mm(wih, xs[t]) + _mm(whh, h) + bb
            i = jax.nn.sigmoid(gt[0:_LH])
            f = jax.nn.sigmoid(gt[_LH:2 * _LH])
            gg = jnp.tanh(gt[2 * _LH:3 * _LH])
            o = jax.nn.sigmoid(gt[3 * _LH:4 * _LH])
            c = f * c + i * gg
            h = o * jnp.tanh(c)
            outs.append(h)
        return outs

    seq = seq_ref[...]
    o1 = lstm([seq[t] for t in range(_T)], wih0_ref[...], whh0_ref[...],
              bb0_ref[...])
    o2 = lstm(o1, wih1_ref[...], whh1_ref[...], bb1_ref[...])
    last = o2[-1]
    hcl = jnp.maximum(_mm(fc1t_ref[...], last) + fc1b_ref[...], 0.0)
    out_ref[...] = _mm(fc2t_ref[...], hcl) + fc2b_ref[...]


def _tc(body, out_shape, *args):
    return pl.pallas_call(body, out_shape=out_shape)(*args)


def kernel(possibility, node_ids, edge_index, batch, emb0, emb1, emb2, emb5,
           W1, b1, W2, b2, W3, b3, g1, be1, g2, be2, g3, be3,
           Wih0, Whh0, bih0, bhh0, Wih1, Whh1, bih1, bhh1,
           fc1_W, fc1_b, fc2_W, fc2_b):
    f32 = jnp.float32
    sds = jax.ShapeDtypeStruct
    possT = jnp.transpose(possibility, (0, 2, 1))
    idsT = jnp.transpose(node_ids, (0, 2, 1)).astype(jnp.int32)
    ei = edge_index.astype(jnp.int32)
    pk = jnp.bitwise_or(ei[:, 0, :], jnp.left_shift(ei[:, 1, :], 16))
    pk_deg = pk.reshape(_T * _NW, _EPT)
    pk_chunks = pk.reshape(_T, _NCHUNK, _CH)
    batch_col = batch.astype(jnp.int32).reshape(_T, _N, 1)
    e0t, e1t, e2t, e5t = (jnp.transpose(e) for e in (emb0, emb1, emb2, emb5))
    w1t, w2t, w3t = (jnp.transpose(w) for w in (W1, W2, W3))
    col = lambda v: v.reshape(-1, 1)

    degp = _deg_kernel(pk_deg)

    feats = []
    for t in range(_T):
        y1, dis = _tc(_k1_body,
                      (sds((_H, _N), f32), sds((1, _N), f32)),
                      possT[t], idsT[t], e0t, e1t, e2t, e5t, w1t,
                      degp[t * _NW:(t + 1) * _NW])
        a1 = _agg_kernel(y1.reshape(-1), pk_chunks[t]).reshape(_H, _N)
        y2 = _tc(_layer_body, sds((_H, _N), f32),
                 a1, dis, col(b1), col(g1), col(be1), w2t)
        a2 = _agg_kernel(y2.reshape(-1), pk_chunks[t]).reshape(_H, _N)
        y3 = _tc(_layer_body, sds((_H, _N), f32),
                 a2, dis, col(b2), col(g2), col(be2), w3t)
        a3 = _agg_kernel(y3.reshape(-1), pk_chunks[t]).reshape(_H, _N)
        ft = _tc(_pool_body, sds((_H, _B), f32),
                 a3, dis, col(b3), col(g3), col(be3), batch_col[t])
        feats.append(ft)

    seq = jnp.stack(feats, 0)
    outT = _tc(_head_body, sds((_NCLS, _B), f32),
               seq, Wih0, Whh0, col(bih0) + col(bhh0),
               Wih1, Whh1, col(bih1) + col(bhh1),
               jnp.transpose(fc1_W), col(fc1_b),
               jnp.transpose(fc2_W), col(fc2_b))
    return jnp.transpose(outT)


# speedup vs baseline: 11.2913x; 1.0803x over previous
import functools

import jax
import jax.numpy as jnp
from jax import lax
from jax.experimental import pallas as pl
from jax.experimental.pallas import tpu as pltpu
from jax.experimental.pallas import tpu_sc as plsc

_T, _N, _E, _B = 4, 10000, 320000, 16
_H, _LH, _NCLS = 256, 128, 3

_NW = 32
_CPT = 4
_ROWS_PER_PASS = _NW * _CPT
_NPASS = _H // _ROWS_PER_PASS
_CH = 6400
_NCHUNK = _E // _CH
_EPT = _E // _NW

_mesh = plsc.VectorSubcoreMesh(core_axis_name="c", subcore_axis_name="s")


def _wid():
    return lax.axis_index("c") * 16 + lax.axis_index("s")


@functools.partial(
    pl.kernel,
    out_type=jax.ShapeDtypeStruct((_T * _NW, _N), jnp.float32),
    mesh=_mesh,
    scratch_types=[
        pltpu.VMEM((_EPT,), jnp.int32),
        pltpu.VMEM((_N,), jnp.float32),
    ],
    compiler_params=pltpu.CompilerParams(needs_layout_passes=False),
)
def _deg_kernel(pk_hbm, out_hbm, pk_v, deg_v):
    w = _wid()
    ones = jnp.ones((16,), jnp.float32)
    zeros = jnp.zeros((16,), jnp.float32)
    for t in range(_T):
        def zbody(i, c):
            deg_v[pl.ds(i * 16, 16)] = zeros
            return c
        lax.fori_loop(0, _N // 16, zbody, 0)
        pltpu.sync_copy(pk_hbm.at[t * _NW + w], pk_v)

        def body(i, c):
            p16 = pk_v[pl.ds(i * 16, 16)]
            d = lax.shift_right_logical(p16, 16)
            plsc.addupdate_scatter(deg_v, [d], ones)
            return c
        lax.fori_loop(0, _EPT // 16, body, 0)
        pltpu.sync_copy(deg_v, out_hbm.at[t * _NW + w])


@functools.partial(
    pl.kernel,
    out_type=jax.ShapeDtypeStruct((_H * _N,), jnp.float32),
    mesh=_mesh,
    scratch_types=[
        pltpu.VMEM((_CPT * _N,), jnp.float32),
        pltpu.VMEM((_CPT * _N,), jnp.float32),
        pltpu.VMEM((2 * _N,), jnp.float32),
        pltpu.VMEM((_CH,), jnp.int32),
        pltpu.VMEM((_CH,), jnp.int32),
        pltpu.SemaphoreType.DMA,
        pltpu.SemaphoreType.DMA,
    ],
    compiler_params=pltpu.CompilerParams(needs_layout_passes=False),
)
def _agg_kernel(y_hbm, pk_hbm, out_hbm, y_v, agg_v, st_v, pk0_v, pk1_v,
                sem0, sem1):
    w = _wid()
    mask = jnp.int32(0xFFFF)
    bufs = ((pk0_v, sem0), (pk1_v, sem1))
    iota = lax.iota(jnp.int32, 16)
    iota4s = iota * 4
    off4 = jnp.bitwise_and(iota, 3)
    pats = [lax.shift_right_logical(iota, 2) + (4 * g) for g in range(4)]

    def perm(x, g):
        return jnp.take_along_axis(x, pats[g], axis=0)

    for p in range(_NPASS):
        row0 = p * _ROWS_PER_PASS + w * _CPT
        for h in range(2):
            pltpu.sync_copy(y_hbm.at[pl.ds((row0 + 2 * h) * _N, 2 * _N)],
                            st_v)

            def il_body(j, c2, h=h):
                c = 2 * h + c2
                v = st_v[pl.ds(c2 * _N + j * 16, 16)]
                idx = iota4s + (j * 64 + c)
                plsc.store_scatter(y_v, [idx], v)
                plsc.store_scatter(agg_v, [idx], v)

            for c2 in range(2):
                @plsc.parallel_loop(0, _N // 16, unroll=8)
                def _il(j, c2=c2):
                    il_body(j, c2)
        pltpu.make_async_copy(pk_hbm.at[0], pk0_v, sem0).start()
        pltpu.make_async_copy(pk_hbm.at[1], pk1_v, sem1).start()

        def outer(g, c):
            for b in range(2):
                pkb, semb = bufs[b]
                gi = g * 2 + b
                pltpu.make_async_copy(pk_hbm.at[gi], pkb, semb).wait()

                @plsc.parallel_loop(0, _CH // 16, unroll=16)
                def _inner(i):
                    p16 = pkb[pl.ds(i * 16, 16)]
                    s4 = jnp.bitwise_and(p16, mask) * 4
                    d4 = lax.shift_right_logical(p16, 16) * 4
                    for g4 in range(4):
                        v = plsc.load_gather(y_v, [perm(s4, g4) + off4])
                        plsc.addupdate_scatter(agg_v, [perm(d4, g4) + off4],
                                               v)

                @pl.when(gi + 2 < _NCHUNK)
                def _start_next():
                    pltpu.make_async_copy(pk_hbm.at[gi + 2], pkb, semb).start()
            return c
        lax.fori_loop(0, _NCHUNK // 2, outer, 0)
        for h in range(2):
            for c2 in range(2):
                @plsc.parallel_loop(0, _N // 16, unroll=8)
                def _dl(j, c2=c2, h=h):
                    c = 2 * h + c2
                    v = plsc.load_gather(agg_v, [iota4s + (j * 64 + c)])
                    st_v[pl.ds(c2 * _N + j * 16, 16)] = v
            pltpu.sync_copy(st_v,
                            out_hbm.at[pl.ds((row0 + 2 * h) * _N, 2 * _N)])


def _mm(a, b):
    return lax.dot_general(a, b, (((1,), (0,)), ((), ())),
                           preferred_element_type=jnp.float32)


def _bn_relu(agg, dis, b, g, be):
    z = agg * dis + b
    m = jnp.mean(z, axis=1, keepdims=True)
    zc = z - m
    v = jnp.mean(zc * zc, axis=1, keepdims=True)
    xh = zc * lax.rsqrt(v + 1e-5) * g + be
    return jnp.maximum(xh, 0.0)


def _k1_body(poss_ref, ids_ref, e0_ref, e1_ref, e2_ref, e5_ref, w1t_ref,
             degp_ref, y_ref, dis_ref):
    deg = jnp.sum(degp_ref[...], axis=0, keepdims=True) + 1.0
    dis = lax.rsqrt(deg)
    dis_ref[...] = dis
    ids = ids_ref[...]
    parts = [poss_ref[...]]
    for row, tref in ((0, e0_ref), (1, e1_ref), (2, e2_ref), (3, e5_ref)):
        tbl = tref[...]
        k = tbl.shape[1]
        iota = lax.broadcasted_iota(jnp.int32, (k, _N), 0)
        oh = (iota == ids[row:row + 1, :]).astype(jnp.float32)
        parts.append(_mm(tbl, oh))
    x0 = jnp.concatenate(parts, axis=0)
    y_ref[...] = _mm(w1t_ref[...], x0) * dis


def _layer_body(agg_ref, dis_ref, b_ref, g_ref, be_ref, wt_ref, y_ref):
    dis = dis_ref[...]
    x = _bn_relu(agg_ref[...], dis, b_ref[...], g_ref[...], be_ref[...])
    y_ref[...] = _mm(wt_ref[...], x) * dis


def _pool_body(agg_ref, dis_ref, b_ref, g_ref, be_ref, batch_ref, feat_ref):
    x = _bn_relu(agg_ref[...], dis_ref[...], b_ref[...], g_ref[...],
                 be_ref[...])
    bt = batch_ref[...]
    iota = lax.broadcasted_iota(jnp.int32, (_N, _B), 1)
    ob = (iota == bt).astype(jnp.float32)
    sums = _mm(x, ob)
    cnt = jnp.sum(ob, axis=0, keepdims=True)
    feat_ref[...] = sums / jnp.maximum(cnt, 1.0)


def _head_body(seq_ref, wih0_ref, whh0_ref, bb0_ref, wih1_ref, whh1_ref,
               bb1_ref, fc1t_ref, fc1b_ref, fc2t_ref, fc2b_ref, out_ref):
    def lstm(xs, wih, whh, bb):
        h = jnp.zeros((_LH, _B), jnp.float32)
        c = jnp.zeros((_LH, _B), jnp.float32)
        outs = []
        for t in range(_T):
            gt = _mm(wih, xs[t]) + _mm(whh, h) + bb
            i = jax.nn.sigmoid(gt[0:_LH])
            f = jax.nn.sigmoid(gt[_LH:2 * _LH])
            gg = jnp.tanh(gt[2 * _LH:3 * _LH])
            o = jax.nn.sigmoid(gt[3 * _LH:4 * _LH])
            c = f * c + i * gg
            h = o * jnp.tanh(c)
            outs.append(h)
        return outs

    seq = seq_ref[...]
    o1 = lstm([seq[t] for t in range(_T)], wih0_ref[...], whh0_ref[...],
              bb0_ref[...])
    o2 = lstm(o1, wih1_ref[...], whh1_ref[...], bb1_ref[...])
    last = o2[-1]
    hcl = jnp.maximum(_mm(fc1t_ref[...], last) + fc1b_ref[...], 0.0)
    out_ref[...] = _mm(fc2t_ref[...], hcl) + fc2b_ref[...]


def _tc(body, out_shape, *args):
    return pl.pallas_call(body, out_shape=out_shape)(*args)


def kernel(possibility, node_ids, edge_index, batch, emb0, emb1, emb2, emb5,
           W1, b1, W2, b2, W3, b3, g1, be1, g2, be2, g3, be3,
           Wih0, Whh0, bih0, bhh0, Wih1, Whh1, bih1, bhh1,
           fc1_W, fc1_b, fc2_W, fc2_b):
    f32 = jnp.float32
    sds = jax.ShapeDtypeStruct
    possT = jnp.transpose(possibility, (0, 2, 1))
    idsT = jnp.transpose(node_ids, (0, 2, 1)).astype(jnp.int32)
    ei = edge_index.astype(jnp.int32)
    pk = jnp.bitwise_or(ei[:, 0, :], jnp.left_shift(ei[:, 1, :], 16))
    pk_deg = pk.reshape(_T * _NW, _EPT)
    pk_chunks = pk.reshape(_T, _NCHUNK, _CH)
    batch_col = batch.astype(jnp.int32).reshape(_T, _N, 1)
    e0t, e1t, e2t, e5t = (jnp.transpose(e) for e in (emb0, emb1, emb2, emb5))
    w1t, w2t, w3t = (jnp.transpose(w) for w in (W1, W2, W3))
    col = lambda v: v.reshape(-1, 1)

    degp = _deg_kernel(pk_deg)

    feats = []
    for t in range(_T):
        y1, dis = _tc(_k1_body,
                      (sds((_H, _N), f32), sds((1, _N), f32)),
                      possT[t], idsT[t], e0t, e1t, e2t, e5t, w1t,
                      degp[t * _NW:(t + 1) * _NW])
        a1 = _agg_kernel(y1.reshape(-1), pk_chunks[t]).reshape(_H, _N)
        y2 = _tc(_layer_body, sds((_H, _N), f32),
                 a1, dis, col(b1), col(g1), col(be1), w2t)
        a2 = _agg_kernel(y2.reshape(-1), pk_chunks[t]).reshape(_H, _N)
        y3 = _tc(_layer_body, sds((_H, _N), f32),
                 a2, dis, col(b2), col(g2), col(be2), w3t)
        a3 = _agg_kernel(y3.reshape(-1), pk_chunks[t]).reshape(_H, _N)
        ft = _tc(_pool_body, sds((_H, _B), f32),
                 a3, dis, col(b3), col(g3), col(be3), batch_col[t])
        feats.append(ft)

    seq = jnp.stack(feats, 0)
    outT = _tc(_head_body, sds((_NCLS, _B), f32),
               seq, Wih0, Whh0, col(bih0) + col(bhh0),
               Wih1, Whh1, col(bih1) + col(bhh1),
               jnp.transpose(fc1_W), col(fc1_b),
               jnp.transpose(fc2_W), col(fc2_b))
    return jnp.transpose(outT)
